# baseline (speedup 1.0000x reference)
import functools

import jax
import jax.numpy as jnp
from jax import lax
from jax.experimental import pallas as pl
from jax.experimental.pallas import tpu as pltpu
from jax.experimental.pallas import tpu_sc as plsc

VOCAB = 50257
D_MODEL = 256
BLOCK = 2048
B = 16
T = 2048

N = B * T
NW = 32
PER_W = N // NW
CH = 128
NCH = PER_W // CH
LANES = 16

_mesh = plsc.VectorSubcoreMesh(core_axis_name="c", subcore_axis_name="s")


@functools.partial(
    pl.kernel,
    mesh=_mesh,
    out_type=jax.ShapeDtypeStruct((N, D_MODEL), jnp.float32),
    scratch_types=[
        pltpu.VMEM((CH,), jnp.int32),
        pltpu.VMEM((CH, D_MODEL), jnp.float32),
        pltpu.VMEM((CH, D_MODEL), jnp.float32),
        pltpu.SemaphoreType.DMA,
    ],
)
def _emb_lookup(x_hbm, tok_hbm, pos_hbm, out_hbm, idx_v, rows_v, pos_v, sem):
    wid = lax.axis_index("s") * 2 + lax.axis_index("c")
    base = wid * PER_W
    t0 = base % T

    for c in range(NCH):
        cb = base + c * CH
        tb = t0 + c * CH
        pltpu.sync_copy(x_hbm.at[pl.ds(cb, CH)], idx_v)
        pltpu.async_copy(tok_hbm.at[idx_v], rows_v, sem).wait()
        pltpu.sync_copy(pos_hbm.at[pl.ds(tb, CH)], pos_v)

        def body(r, carry):
            for j in range(D_MODEL // LANES):
                sl = pl.ds(j * LANES, LANES)
                rows_v[r, sl] = rows_v[r, sl] + pos_v[r, sl]
            return carry

        lax.fori_loop(0, CH, body, 0)
        pltpu.sync_copy(rows_v, out_hbm.at[pl.ds(cb, CH)])


def kernel(x, token_table, pos_table):
    xf = x.reshape(-1).astype(jnp.int32)
    out = _emb_lookup(xf, token_table, pos_table)
    return out.reshape(B, T, D_MODEL)


# speedup vs baseline: 1.6217x; 1.6217x over previous
import functools

import jax
import jax.numpy as jnp
from jax import lax
from jax.experimental import pallas as pl
from jax.experimental.pallas import tpu as pltpu
from jax.experimental.pallas import tpu_sc as plsc

D_MODEL = 256
B = 16
T = 2048

N = B * T
NW = 32
TW = T // NW
LANES = 16
NVEC = D_MODEL // LANES
NBUF = 4
DEPTH = 3

_mesh = plsc.VectorSubcoreMesh(core_axis_name="c", subcore_axis_name="s")


@functools.partial(
    pl.kernel,
    mesh=_mesh,
    out_type=jax.ShapeDtypeStruct((N, D_MODEL), jnp.float32),
    scratch_types=[
        pltpu.VMEM((B, TW), jnp.int32),
        pltpu.VMEM((TW, D_MODEL), jnp.float32),
    ]
    + [pltpu.VMEM((TW, D_MODEL), jnp.float32) for _ in range(NBUF)]
    + [pltpu.SemaphoreType.DMA for _ in range(2 * NBUF)],
)
def _emb_lookup(x_hbm, tok_hbm, pos_hbm, out_hbm, idx_v, pos_v, *rest):
    bufs = list(rest[:NBUF])
    gsems = list(rest[NBUF : 2 * NBUF])
    osems = list(rest[2 * NBUF : 3 * NBUF])

    wid = lax.axis_index("s") * 2 + lax.axis_index("c")
    t0 = wid * TW

    for b in range(B):
        pltpu.sync_copy(x_hbm.at[pl.ds(b * T + t0, TW)], idx_v.at[b])
    pltpu.sync_copy(pos_hbm.at[pl.ds(t0, TW)], pos_v)

    def gather(b):
        s = b % NBUF
        return pltpu.async_copy(tok_hbm.at[idx_v.at[b]], bufs[s], gsems[s])

    gd = {}
    od = {}
    for b in range(DEPTH):
        gd[b % NBUF] = gather(b)

    for b in range(B):
        s = b % NBUF
        gd.pop(s).wait()

        buf = bufs[s]

        def add_row(r, carry):
            for j in range(NVEC):
                sl = pl.ds(j * LANES, LANES)
                plsc.addupdate(buf.at[r, sl], pos_v[r, sl])
            return carry

        lax.fori_loop(0, TW, add_row, 0)

        od[s] = pltpu.async_copy(buf, out_hbm.at[pl.ds(b * T + t0, TW)], osems[s])

        nb = b + DEPTH
        if nb < B:
            ns = nb % NBUF
            if ns in od:
                od.pop(ns).wait()
            gd[ns] = gather(nb)

    for s in sorted(od):
        od.pop(s).wait()


def kernel(x, token_table, pos_table):
    xf = x.reshape(-1).astype(jnp.int32)
    out = _emb_lookup(xf, token_table, pos_table)
    return out.reshape(B, T, D_MODEL)
